# Initial kernel scaffold; baseline (speedup 1.0000x reference)
import jax
import jax.numpy as jnp
from jax.experimental import pallas as pl


def kernel(node_feats, edge_feats, edge_index, Wq, bq, Wk, bk, Wv, bv, We, be):
    raise NotImplementedError("write your pallas kernel here")


# speedup vs baseline: 13.0461x; 13.0461x over previous
import functools

import jax
import jax.numpy as jnp
from jax import lax
from jax.experimental import pallas as pl
from jax.experimental.pallas import tpu as pltpu
from jax.experimental.pallas import tpu_sc as plsc

N = 10000
E = 320000
D_IN = 128
H = 8
D = 16
HD = H * D
CLIP = 5.0

NC = 2
NS = 16
NW = NC * NS
C = 40
EPW = E // NW
CHUNKS = EPW // C
ACC_W = HD + 16
RCHUNK = 40
NRC = N // RCHUNK
NRC_PT = -(-NRC // NS)


def _matmul_bias(x, w, b, block_rows):
    m, k = x.shape
    f = w.shape[1]

    def body(x_ref, w_ref, b_ref, o_ref):
        o_ref[...] = (
            jnp.dot(x_ref[...], w_ref[...], preferred_element_type=jnp.float32)
            + b_ref[0:1, :]
        )

    return pl.pallas_call(
        body,
        grid=(m // block_rows,),
        in_specs=[
            pl.BlockSpec((block_rows, k), lambda i: (i, 0)),
            pl.BlockSpec((k, f), lambda i: (0, 0)),
            pl.BlockSpec((8, f), lambda i: (0, 0)),
        ],
        out_specs=pl.BlockSpec((block_rows, f), lambda i: (i, 0)),
        out_shape=jax.ShapeDtypeStruct((m, f), jnp.float32),
    )(x, w, jnp.broadcast_to(b, (8, f)))


def _combine(wvz_parts):
    bn = 2000

    def body(p_ref, o_ref):
        p = p_ref[0] + p_ref[1]
        wv = p[:, :HD]
        z = p[:, HD:]
        col = lax.broadcasted_iota(jnp.int32, (16, HD), 1)
        row = lax.broadcasted_iota(jnp.int32, (16, HD), 0)
        sel = ((col // D == row) & (row < H)).astype(jnp.float32)
        zrep = jnp.dot(z, sel, preferred_element_type=jnp.float32)
        o_ref[...] = wv / (zrep + 1e-6)

    return pl.pallas_call(
        body,
        grid=(N // bn,),
        in_specs=[
            pl.BlockSpec((NC, bn, ACC_W), lambda i: (0, i, 0)),
        ],
        out_specs=pl.BlockSpec((bn, HD), lambda i: (i, 0)),
        out_shape=jax.ShapeDtypeStruct((N, HD), jnp.float32),
    )(wvz_parts)


@functools.lru_cache(maxsize=1)
def _make_sc_edge_kernel():
    mesh = plsc.VectorSubcoreMesh(
        core_axis_name="c", subcore_axis_name="s",
        num_cores=NC, num_subcores=NS)

    @functools.partial(
        pl.kernel,
        out_type=[
            jax.ShapeDtypeStruct((E, HD), jnp.float32),
            jax.ShapeDtypeStruct((NC, N, ACC_W), jnp.float32),
        ],
        mesh=mesh,
        compiler_params=pltpu.CompilerParams(use_tc_tiling_on_sc=False),
        scratch_types=[
            pltpu.VMEM((C,), jnp.int32),
            pltpu.VMEM((C,), jnp.int32),
            pltpu.VMEM((C, HD), jnp.float32),
            pltpu.VMEM((C, HD), jnp.float32),
            pltpu.VMEM((C, HD), jnp.float32),
            pltpu.VMEM((C, HD), jnp.float32),
            pltpu.VMEM((C, HD), jnp.float32),
            pltpu.VMEM((C, ACC_W), jnp.float32),
            pltpu.VMEM((RCHUNK, ACC_W), jnp.float32),
            pltpu.VMEM_SHARED((N, ACC_W), jnp.float32),
            pltpu.SemaphoreType.DMA,
            pltpu.SemaphoreType.DMA,
            pltpu.SemaphoreType.DMA,
        ],
    )
    def sc_edge_kernel(kh, qh, vh, pe, src, dst,
                       eout_hbm, wv_hbm,
                       src_v, dst_v, k_v, q_v, v_v, pe_v, eo_v, wv_v,
                       zb, acc, sem_k, sem_q, sem_v):
        cid = lax.axis_index("c")
        sid = lax.axis_index("s")
        wid = cid * NS + sid

        zeros16 = jnp.zeros((16,), jnp.float32)

        def zero_row(r, _):
            for h in range(ACC_W // D):
                zb[r, pl.ds(h * D, D)] = zeros16
            return 0

        lax.fori_loop(0, RCHUNK, zero_row, 0)

        def init_body(j, _):
            row = j * NS + sid

            @pl.when(row < NRC)
            def _():
                base = pl.multiple_of(row * RCHUNK, 8)
                pltpu.sync_copy(zb, acc.at[pl.ds(base, RCHUNK), :])
            return 0

        lax.fori_loop(0, NRC_PT, init_body, 0)
        plsc.subcore_barrier()

        lane = lax.iota(jnp.int32, 16)

        def chunk_body(i, _):
            eb = pl.multiple_of(wid * EPW + i * C, 8)
            pltpu.sync_copy(src.at[pl.ds(eb, C)], src_v)
            pltpu.sync_copy(dst.at[pl.ds(eb, C)], dst_v)
            pltpu.sync_copy(pe.at[pl.ds(eb, C), :], pe_v)
            cp_k = pltpu.async_copy(kh.at[src_v], k_v, sem_k)
            cp_q = pltpu.async_copy(qh.at[dst_v], q_v, sem_q)
            cp_v = pltpu.async_copy(vh.at[src_v], v_v, sem_v)
            cp_k.wait()
            cp_q.wait()
            cp_v.wait()

            def edge_body(c, _):
                zrow = zeros16
                for h in range(H):
                    sl = pl.ds(h * D, D)
                    kq = k_v[c, sl] * q_v[c, sl]
                    s1 = jnp.clip(kq * 0.25, -CLIP, CLIP)
                    s = s1 * pe_v[c, sl]
                    eo_v[c, sl] = s
                    sv = s
                    for sh in (8, 4, 2, 1):
                        sv = sv + sv.at[lane ^ sh].get(
                            mode="promise_in_bounds")
                    tv = jnp.exp(jnp.clip(sv, -CLIP, CLIP))
                    wv_v[c, sl] = v_v[c, sl] * tv
                    zrow = jnp.where(lane == h, tv, zrow)
                wv_v[c, pl.ds(HD, 16)] = zrow
                return 0

            lax.fori_loop(0, C, edge_body, 0)
            pltpu.sync_copy(eo_v, eout_hbm.at[pl.ds(eb, C), :])
            pltpu.sync_copy(wv_v, acc.at[dst_v], add=True)
            return 0

        lax.fori_loop(0, CHUNKS, chunk_body, 0)
        plsc.subcore_barrier()

        def out_body(j, _):
            row = j * NS + sid

            @pl.when(row < NRC)
            def _():
                base = pl.multiple_of(row * RCHUNK, 8)
                pltpu.sync_copy(acc.at[pl.ds(base, RCHUNK), :], zb)
                pltpu.sync_copy(zb, wv_hbm.at[cid, pl.ds(base, RCHUNK), :])
            return 0

        lax.fori_loop(0, NRC_PT, out_body, 0)

    return sc_edge_kernel


def kernel(node_feats, edge_feats, edge_index, Wq, bq, Wk, bk, Wv, bv, We, be):
    w_qkv = jnp.concatenate([Wq, Wk, Wv], axis=1)
    b_qkv = jnp.concatenate([bq, bk, bv], axis=0)
    qkv = _matmul_bias(node_feats, w_qkv, b_qkv, 2000)
    q_h = qkv[:, :HD]
    k_h = qkv[:, HD:2 * HD]
    v_h = qkv[:, 2 * HD:]
    pe = _matmul_bias(edge_feats, We, be, 5000)

    src = edge_index[0]
    dst = edge_index[1]
    e_out, wvz_parts = _make_sc_edge_kernel()(k_h, q_h, v_h, pe, src, dst)
    h_out = _combine(wvz_parts)
    return (h_out.reshape(N, H, D), e_out.reshape(E, H, D))
